# baseline (speedup 1.0000x reference)
import functools

import jax
import jax.numpy as jnp
from jax import lax
from jax.experimental import pallas as pl
from jax.experimental.pallas import tpu as pltpu
from jax.experimental.pallas import tpu_sc as plsc

_LANES = 128


def _make_sc_gather(n_rows, lanes, d_model, n_cores, n_subcores):
    n_workers = n_cores * n_subcores
    rows_per_w = n_rows // n_workers

    mesh = plsc.VectorSubcoreMesh(core_axis_name="c", subcore_axis_name="s")

    @functools.partial(
        pl.kernel,
        mesh=mesh,
        out_type=jax.ShapeDtypeStruct((n_rows * lanes, d_model), jnp.float32),
        scratch_types=[
            pltpu.VMEM((lanes,), jnp.int32),
            pltpu.VMEM((lanes, d_model), jnp.float32),
            pltpu.SemaphoreType.DMA,
        ],
        compiler_params=pltpu.CompilerParams(use_tc_tiling_on_sc=False),
    )
    def k(ids_hbm, table_hbm, out_hbm, idx_v, rows_v, sem):
        wid = lax.axis_index("s") * n_cores + lax.axis_index("c")
        row0 = wid * rows_per_w

        def body(i, carry):
            r = row0 + i
            pltpu.sync_copy(ids_hbm.at[r], idx_v)
            pltpu.async_copy(table_hbm.at[idx_v], rows_v, sem).wait()
            pltpu.sync_copy(rows_v, out_hbm.at[pl.ds(r * lanes, lanes)])
            return carry

        lax.fori_loop(0, rows_per_w, body, 0)

    return k


def kernel(token_ids, weights):
    batch, seq_len = token_ids.shape
    vocab, d_model = weights.shape
    n = batch * seq_len
    n_rows = n // _LANES
    ids2d = token_ids.astype(jnp.int32).reshape(n_rows, _LANES)

    info = plsc.get_sparse_core_info()
    k = _make_sc_gather(n_rows, _LANES, d_model, info.num_cores, info.num_subcores)
    out = k(ids2d, weights)
    return out.reshape(batch, seq_len, d_model)


# speedup vs baseline: 1.1917x; 1.1917x over previous
import functools

import jax
import jax.numpy as jnp
from jax import lax
from jax.experimental import pallas as pl
from jax.experimental.pallas import tpu as pltpu
from jax.experimental.pallas import tpu_sc as plsc

_LANES = 128
_NBUF = 4


def _make_sc_gather(n_rows, lanes, d_model, n_cores, n_subcores):
    n_workers = n_cores * n_subcores
    rows_per_w = n_rows // n_workers
    n_groups = rows_per_w // _NBUF
    assert n_groups % 2 == 0 and n_groups >= 4

    mesh = plsc.VectorSubcoreMesh(core_axis_name="c", subcore_axis_name="s")

    @functools.partial(
        pl.kernel,
        mesh=mesh,
        out_type=jax.ShapeDtypeStruct((n_rows * lanes, d_model), jnp.float32),
        scratch_types=[
            pltpu.VMEM((rows_per_w, lanes), jnp.int32),
            pltpu.VMEM((2, _NBUF, lanes, d_model), jnp.float32),
            pltpu.SemaphoreType.DMA((2, _NBUF)),
            pltpu.SemaphoreType.DMA((2, _NBUF)),
        ],
        compiler_params=pltpu.CompilerParams(use_tc_tiling_on_sc=False),
    )
    def k(ids_hbm, table_hbm, out_hbm, idx_v, rows_v, sem_g, sem_s):
        wid = lax.axis_index("s") * n_cores + lax.axis_index("c")
        row0 = wid * rows_per_w
        pltpu.sync_copy(ids_hbm.at[pl.ds(row0, rows_per_w)], idx_v)

        def gather_copy(g, h, b):
            return pltpu.make_async_copy(
                table_hbm.at[idx_v.at[g * _NBUF + b]],
                rows_v.at[h, b],
                sem_g.at[h, b],
            )

        def store_copy(g, h, b):
            return pltpu.make_async_copy(
                rows_v.at[h, b],
                out_hbm.at[pl.ds((row0 + g * _NBUF + b) * lanes, lanes)],
                sem_s.at[h, b],
            )

        def process(g, h, first=False, last=False):
            nh = 1 - h
            if not first:
                for b in range(_NBUF):
                    store_copy(g - 1, nh, b).wait()
            if not last:
                for b in range(_NBUF):
                    gather_copy(g + 1, nh, b).start()
            for b in range(_NBUF):
                gather_copy(g, h, b).wait()
                store_copy(g, h, b).start()

        for b in range(_NBUF):
            gather_copy(0, 0, b).start()
        process(0, 0, first=True)
        process(1, 1)

        def body(t, carry):
            g2 = 2 * t
            process(g2, 0)
            process(g2 + 1, 1)
            return carry

        lax.fori_loop(1, n_groups // 2 - 1, body, 0)

        process(n_groups - 2, 0)
        process(n_groups - 1, 1, last=True)
        for b in range(_NBUF):
            store_copy(n_groups - 1, 1, b).wait()

    return k


def kernel(token_ids, weights):
    batch, seq_len = token_ids.shape
    vocab, d_model = weights.shape
    n = batch * seq_len
    n_rows = n // _LANES
    ids2d = token_ids.astype(jnp.int32).reshape(n_rows, _LANES)

    info = plsc.get_sparse_core_info()
    k = _make_sc_gather(n_rows, _LANES, d_model, info.num_cores, info.num_subcores)
    out = k(ids2d, weights)
    return out.reshape(batch, seq_len, d_model)


# speedup vs baseline: 1.4538x; 1.2199x over previous
import functools

import jax
import jax.numpy as jnp
from jax import lax
from jax.experimental import pallas as pl
from jax.experimental.pallas import tpu as pltpu
from jax.experimental.pallas import tpu_sc as plsc

_LANES = 128
_NBUF = 2


def _make_sc_gather(n_rows, lanes, width, n_cores, n_subcores):
    n_workers = n_cores * n_subcores
    rows_per_w = n_rows // n_workers
    n_groups = rows_per_w // _NBUF
    assert n_groups % 2 == 0 and n_groups >= 4

    mesh = plsc.VectorSubcoreMesh(core_axis_name="c", subcore_axis_name="s")

    @functools.partial(
        pl.kernel,
        mesh=mesh,
        out_type=jax.ShapeDtypeStruct((n_rows * lanes, width), jnp.float32),
        scratch_types=[
            pltpu.VMEM((rows_per_w, lanes), jnp.int32),
            pltpu.VMEM((2, _NBUF, lanes, width), jnp.float32),
            pltpu.SemaphoreType.DMA((2, _NBUF)),
            pltpu.SemaphoreType.DMA((2, _NBUF)),
        ],
        compiler_params=pltpu.CompilerParams(use_tc_tiling_on_sc=False),
    )
    def k(ids_hbm, table_hbm, out_hbm, idx_v, rows_v, sem_g, sem_s):
        wid = lax.axis_index("s") * n_cores + lax.axis_index("c")
        row0 = wid * rows_per_w
        pltpu.sync_copy(ids_hbm.at[pl.ds(row0, rows_per_w)], idx_v)

        def gather_copy(g, h, b):
            return pltpu.make_async_copy(
                table_hbm.at[idx_v.at[g * _NBUF + b]],
                rows_v.at[h, b],
                sem_g.at[h, b],
            )

        def store_copy(g, h, b):
            return pltpu.make_async_copy(
                rows_v.at[h, b],
                out_hbm.at[pl.ds((row0 + g * _NBUF + b) * lanes, lanes)],
                sem_s.at[h, b],
            )

        def process(g, h, first=False, last=False):
            nh = 1 - h
            if not first:
                for b in range(_NBUF):
                    store_copy(g - 1, nh, b).wait()
            if not last:
                for b in range(_NBUF):
                    gather_copy(g + 1, nh, b).start()
            for b in range(_NBUF):
                gather_copy(g, h, b).wait()
                store_copy(g, h, b).start()

        for b in range(_NBUF):
            gather_copy(0, 0, b).start()
        process(0, 0, first=True)
        process(1, 1)

        def body(t, carry):
            g2 = 2 * t
            process(g2, 0)
            process(g2 + 1, 1)
            return carry

        lax.fori_loop(1, n_groups // 2 - 1, body, 0)

        process(n_groups - 2, 0)
        process(n_groups - 1, 1, last=True)
        for b in range(_NBUF):
            store_copy(n_groups - 1, 1, b).wait()

    return k


def kernel(token_ids, weights):
    batch, seq_len = token_ids.shape
    vocab, d_model = weights.shape
    n = batch * seq_len
    n_rows = n // _LANES
    ids2d = token_ids.astype(jnp.int32).reshape(n_rows, _LANES)
    wp = jnp.pad(weights, ((0, 0), (0, _LANES - d_model)))

    info = plsc.get_sparse_core_info()
    k = _make_sc_gather(n_rows, _LANES, _LANES, info.num_cores, info.num_subcores)
    out = k(ids2d, wp)
    return out[:, :d_model].reshape(batch, seq_len, d_model)
